# baseline (speedup 1.0000x reference)
import functools
import jax
import jax.numpy as jnp
from jax import lax
from jax.experimental import pallas as pl
from jax.experimental.pallas import tpu as pltpu, tpu_sc as plsc

H = 224
W = 224
HO = H // 2
WO = W // 2
NV = W // 16
NB = 256
NBV = NB // 16
IMGS = 768
IMGS_PER_WORKER = IMGS // 32
CHUNK = 8
UNROLL = 2

_MESH = plsc.VectorSubcoreMesh(core_axis_name="c", subcore_axis_name="s")


def _sortable_i32(xf):
    xb = plsc.bitcast(xf, jnp.int32)
    flip = (xb >> 31) | jnp.int32(-(2**31))
    return xb ^ flip


def _digit(key_i, shift):
    d = (plsc.bitcast(key_i, jnp.uint32) >> jnp.uint32(shift)) & jnp.uint32(NB - 1)
    return plsc.bitcast(d, jnp.int32)


def _body(x_hbm, out_hbm, ximg, oimg, kAs, kBs, vAs, vBs, hists, bases, srows):
    wid = lax.axis_index("s") * 2 + lax.axis_index("c")
    iota = lax.iota(jnp.int32, 16)
    ones = jnp.ones((16,), jnp.int32)

    def sort_row(h, r):
        ro = r * W
        rh = r * NB

        def hist_zero():
            for i in range(NBV):
                hists[pl.ds(rh + 16 * i, 16)] = jnp.zeros((16,), jnp.int32)

        def scan_bases():
            carry = jnp.int32(0)
            for i in range(NBV):
                hv = hists[pl.ds(rh + 16 * i, 16)]
                c = plsc.cumsum(hv)
                bases[pl.ds(rh + 16 * i, 16)] = c - hv + carry
                carry = carry + c[15]

        hist_zero()
        for v in range(NV):
            key = _sortable_i32(ximg[h, pl.ds(16 * v, 16)])
            kAs[pl.ds(ro + 16 * v, 16)] = key
            plsc.addupdate_scatter(hists, [rh + _digit(key, 0)], ones)
        scan_bases()
        for v in range(NV):
            k = kAs[pl.ds(ro + 16 * v, 16)]
            d = _digit(k, 0)
            rc, last = plsc.scan_count(d)
            pos = plsc.load_gather(bases, [rh + d]) + rc - 1
            plsc.store_scatter(kBs, [ro + pos], k)
            plsc.store_scatter(vBs, [ro + pos], iota + jnp.int32(16 * v))
            plsc.addupdate_scatter(bases, [rh + d], rc, mask=last)
        for p, (sk, sv, dk, dv) in enumerate(
            [(kBs, vBs, kAs, vAs), (kAs, vAs, kBs, vBs), (kBs, vBs, kAs, vAs)],
            start=1,
        ):
            shift = 8 * p
            hist_zero()
            for v in range(NV):
                plsc.addupdate_scatter(
                    hists, [rh + _digit(sk[pl.ds(ro + 16 * v, 16)], shift)], ones)
            scan_bases()
            for v in range(NV):
                k = sk[pl.ds(ro + 16 * v, 16)]
                val = sv[pl.ds(ro + 16 * v, 16)]
                d = _digit(k, shift)
                rc, last = plsc.scan_count(d)
                pos = plsc.load_gather(bases, [rh + d]) + rc - 1
                if p < 3:
                    plsc.store_scatter(dk, [ro + pos], k)
                plsc.store_scatter(dv, [ro + pos], val)
                plsc.addupdate_scatter(bases, [rh + d], rc, mask=last)
        rs = r * WO
        for m in range(WO // 16):
            idx = iota * 2 + jnp.int32(ro + 32 * m)
            e = plsc.load_gather(vAs, [idx])
            o = plsc.load_gather(vAs, [idx + 1])
            srows[pl.ds(rs + 16 * m, 16)] = e + o

    def img_body(jj, _):
        img = wid * IMGS_PER_WORKER + jj
        pltpu.sync_copy(x_hbm.at[pl.ds(img * H, H)], ximg)

        def group_body(g, _):
            @plsc.parallel_loop(0, CHUNK, unroll=UNROLL)
            def row_loop(r):
                sort_row(CHUNK * g + r, r)

            for pr in range(CHUNK // 2):
                hp = (CHUNK // 2) * g + pr
                for m in range(WO // 16):
                    tot = (
                        srows[pl.ds(2 * pr * WO + 16 * m, 16)]
                        + srows[pl.ds((2 * pr + 1) * WO + 16 * m, 16)]
                    ).astype(jnp.float32) * 0.25
                    oimg[pl.ds(hp * WO + 16 * m, 16)] = tot
            return 0

        lax.fori_loop(0, H // CHUNK, group_body, 0)
        pltpu.sync_copy(oimg, out_hbm.at[img])
        return 0

    lax.fori_loop(0, IMGS_PER_WORKER, img_body, 0)


@functools.partial(
    pl.kernel,
    out_type=jax.ShapeDtypeStruct((IMGS, HO * WO), jnp.float32),
    mesh=_MESH,
    compiler_params=pltpu.CompilerParams(needs_layout_passes=False),
    scratch_types=[
        pltpu.VMEM((H, W), jnp.float32),
        pltpu.VMEM((HO * WO,), jnp.float32),
        pltpu.VMEM((CHUNK * W,), jnp.int32),
        pltpu.VMEM((CHUNK * W,), jnp.int32),
        pltpu.VMEM((CHUNK * W,), jnp.int32),
        pltpu.VMEM((CHUNK * W,), jnp.int32),
        pltpu.VMEM((CHUNK * NB,), jnp.int32),
        pltpu.VMEM((CHUNK * NB,), jnp.int32),
        pltpu.VMEM((CHUNK * WO,), jnp.int32),
    ],
)
def _sc_kernel(x_hbm, out_hbm, *scratch):
    _body(x_hbm, out_hbm, *scratch)


@jax.jit
def kernel(x):
    b, c, h, w = x.shape
    xf = x.reshape(b * c * h, w)
    out = _sc_kernel(xf)
    return out.reshape(b, c, HO, WO)


# speedup vs baseline: 1.1599x; 1.1599x over previous
import functools
import jax
import jax.numpy as jnp
from jax import lax
from jax.experimental import pallas as pl
from jax.experimental.pallas import tpu as pltpu, tpu_sc as plsc

H = 224
W = 224
HO = H // 2
WO = W // 2
NV = W // 16
NB = 256
NBV = NB // 16
IMGS = 768
IMGS_PER_WORKER = IMGS // 32

_MESH = plsc.VectorSubcoreMesh(core_axis_name="c", subcore_axis_name="s")


def _sortable_i32(xf):
    xb = plsc.bitcast(xf, jnp.int32)
    flip = (xb >> 31) | jnp.int32(-(2**31))
    return xb ^ flip


def _digit(w_i, shift):
    d = (plsc.bitcast(w_i, jnp.uint32) >> jnp.uint32(shift)) & jnp.uint32(NB - 1)
    return plsc.bitcast(d, jnp.int32)


def _body(x_hbm, out_hbm, ximg, oimg, srow,
          wA0, wB0, hist0, base0, wA1, wB1, hist1, base1):
    wid = lax.axis_index("s") * 2 + lax.axis_index("c")
    iota = lax.iota(jnp.int32, 16)
    ones = jnp.ones((16,), jnp.int32)

    def sort_row(h, wA, wB, hist, base):
        def hist_zero():
            for i in range(NBV):
                hist[pl.ds(16 * i, 16)] = jnp.zeros((16,), jnp.int32)

        def scan_bases():
            carry = jnp.int32(0)
            for i in range(NBV):
                hv = hist[pl.ds(16 * i, 16)]
                c = plsc.cumsum(hv)
                base[pl.ds(16 * i, 16)] = c - hv + carry
                carry = carry + c[15]

        hist_zero()
        for v in range(NV):
            key = _sortable_i32(ximg[h, pl.ds(16 * v, 16)])
            plsc.addupdate_scatter(hist, [_digit(key, 0)], ones)
        scan_bases()
        for v in range(NV):
            key = _sortable_i32(ximg[h, pl.ds(16 * v, 16)])
            d = _digit(key, 0)
            w = (key & jnp.int32(-256)) | (iota + jnp.int32(16 * v))
            rc, last = plsc.scan_count(d)
            pos = plsc.load_gather(base, [d]) + rc - 1
            plsc.store_scatter(wB, [pos], w)
            plsc.addupdate_scatter(base, [d], rc, mask=last)
        for p, (src, dst) in enumerate([(wB, wA), (wA, wB)], start=1):
            shift = 8 * p
            hist_zero()
            for v in range(NV):
                plsc.addupdate_scatter(
                    hist, [_digit(src[pl.ds(16 * v, 16)], shift)], ones)
            scan_bases()
            for v in range(NV):
                w = src[pl.ds(16 * v, 16)]
                d = _digit(w, shift)
                rc, last = plsc.scan_count(d)
                pos = plsc.load_gather(base, [d]) + rc - 1
                plsc.store_scatter(dst, [pos], w)
                plsc.addupdate_scatter(base, [d], rc, mask=last)
        hist_zero()
        for v in range(NV):
            plsc.addupdate_scatter(
                hist, [_digit(wB[pl.ds(16 * v, 16)], 24)], ones)
        scan_bases()
        for v in range(NV):
            w = wB[pl.ds(16 * v, 16)]
            d = _digit(w, 24)
            rc, last = plsc.scan_count(d)
            pos = plsc.load_gather(base, [d]) + rc - 1
            plsc.addupdate_scatter(srow, [pos >> 1], w & jnp.int32(255))
            plsc.addupdate_scatter(base, [d], rc, mask=last)

    def img_body(jj, _):
        img = wid * IMGS_PER_WORKER + jj
        pltpu.sync_copy(x_hbm.at[pl.ds(img * H, H)], ximg)

        def pair_body(hp, _):
            for m in range(WO // 16):
                srow[pl.ds(16 * m, 16)] = jnp.zeros((16,), jnp.int32)
            sort_row(2 * hp, wA0, wB0, hist0, base0)
            sort_row(2 * hp + 1, wA1, wB1, hist1, base1)
            for m in range(WO // 16):
                oimg[pl.ds(hp * WO + 16 * m, 16)] = (
                    srow[pl.ds(16 * m, 16)].astype(jnp.float32) * 0.25)
            return 0

        lax.fori_loop(0, HO, pair_body, 0)
        pltpu.sync_copy(oimg, out_hbm.at[img])
        return 0

    lax.fori_loop(0, IMGS_PER_WORKER, img_body, 0)


@functools.partial(
    pl.kernel,
    out_type=jax.ShapeDtypeStruct((IMGS, HO * WO), jnp.float32),
    mesh=_MESH,
    compiler_params=pltpu.CompilerParams(needs_layout_passes=False),
    scratch_types=[
        pltpu.VMEM((H, W), jnp.float32),
        pltpu.VMEM((HO * WO,), jnp.float32),
        pltpu.VMEM((WO,), jnp.int32),
        pltpu.VMEM((W,), jnp.int32),
        pltpu.VMEM((W,), jnp.int32),
        pltpu.VMEM((NB,), jnp.int32),
        pltpu.VMEM((NB,), jnp.int32),
        pltpu.VMEM((W,), jnp.int32),
        pltpu.VMEM((W,), jnp.int32),
        pltpu.VMEM((NB,), jnp.int32),
        pltpu.VMEM((NB,), jnp.int32),
    ],
)
def _sc_kernel(x_hbm, out_hbm, *scratch):
    _body(x_hbm, out_hbm, *scratch)


@jax.jit
def kernel(x):
    b, c, h, w = x.shape
    xf = x.reshape(b * c * h, w)
    out = _sc_kernel(xf)
    return out.reshape(b, c, HO, WO)


# speedup vs baseline: 1.7436x; 1.5032x over previous
import functools
import jax
import jax.numpy as jnp
from jax import lax
from jax.experimental import pallas as pl
from jax.experimental.pallas import tpu as pltpu, tpu_sc as plsc

H = 224
W = 224
HO = H // 2
WO = W // 2
NV = W // 16
NB = 256
NBV = NB // 16
IMGS = 768
IMGS_PER_WORKER = IMGS // 32

_MESH = plsc.VectorSubcoreMesh(core_axis_name="c", subcore_axis_name="s")


def _sortable_i32(xf):
    xb = plsc.bitcast(xf, jnp.int32)
    flip = (xb >> 31) | jnp.int32(-(2**31))
    return xb ^ flip


def _digit(w_i, shift):
    d = (plsc.bitcast(w_i, jnp.uint32) >> jnp.uint32(shift)) & jnp.uint32(NB - 1)
    return plsc.bitcast(d, jnp.int32)


def _body(x_hbm, out_hbm, ximg, oimg, srow,
          wA0, wB0, hist0, base0, wA1, wB1, hist1, base1):
    wid = lax.axis_index("s") * 2 + lax.axis_index("c")
    iota = lax.iota(jnp.int32, 16)
    ones = jnp.ones((16,), jnp.int32)
    zeros = jnp.zeros((16,), jnp.int32)

    R0 = (wA0, wB0, hist0, base0)
    R1 = (wA1, wB1, hist1, base1)

    def sort_pair(h0):
        def hist_zero():
            for i in range(NBV):
                hist0[pl.ds(16 * i, 16)] = zeros
                hist1[pl.ds(16 * i, 16)] = zeros

        def scan_bases():
            carry0 = jnp.int32(0)
            carry1 = jnp.int32(0)
            for i in range(NBV):
                hv0 = hist0[pl.ds(16 * i, 16)]
                hv1 = hist1[pl.ds(16 * i, 16)]
                c0 = plsc.cumsum(hv0)
                c1 = plsc.cumsum(hv1)
                base0[pl.ds(16 * i, 16)] = c0 - hv0 + carry0
                base1[pl.ds(16 * i, 16)] = c1 - hv1 + carry1
                carry0 = carry0 + c0[15]
                carry1 = carry1 + c1[15]

        hist_zero()
        for v in range(NV):
            key0 = _sortable_i32(ximg[h0, pl.ds(16 * v, 16)])
            key1 = _sortable_i32(ximg[h0 + 1, pl.ds(16 * v, 16)])
            plsc.addupdate_scatter(hist0, [_digit(key0, 0)], ones)
            plsc.addupdate_scatter(hist1, [_digit(key1, 0)], ones)
        scan_bases()
        for v in range(NV):
            key0 = _sortable_i32(ximg[h0, pl.ds(16 * v, 16)])
            key1 = _sortable_i32(ximg[h0 + 1, pl.ds(16 * v, 16)])
            d0 = _digit(key0, 0)
            d1 = _digit(key1, 0)
            lane = iota + jnp.int32(16 * v)
            w0 = (key0 & jnp.int32(-256)) | lane
            w1 = (key1 & jnp.int32(-256)) | lane
            rc0, last0 = plsc.scan_count(d0)
            rc1, last1 = plsc.scan_count(d1)
            pos0 = plsc.load_gather(base0, [d0]) + rc0 - 1
            pos1 = plsc.load_gather(base1, [d1]) + rc1 - 1
            plsc.store_scatter(wB0, [pos0], w0)
            plsc.store_scatter(wB1, [pos1], w1)
            plsc.addupdate_scatter(base0, [d0], rc0, mask=last0)
            plsc.addupdate_scatter(base1, [d1], rc1, mask=last1)
        for p, (s0, d0_, s1, d1_) in enumerate(
            [(wB0, wA0, wB1, wA1), (wA0, wB0, wA1, wB1)], start=1
        ):
            shift = 8 * p
            hist_zero()
            for v in range(NV):
                g0 = _digit(s0[pl.ds(16 * v, 16)], shift)
                g1 = _digit(s1[pl.ds(16 * v, 16)], shift)
                plsc.addupdate_scatter(hist0, [g0], ones)
                plsc.addupdate_scatter(hist1, [g1], ones)
            scan_bases()
            for v in range(NV):
                w0 = s0[pl.ds(16 * v, 16)]
                w1 = s1[pl.ds(16 * v, 16)]
                g0 = _digit(w0, shift)
                g1 = _digit(w1, shift)
                rc0, last0 = plsc.scan_count(g0)
                rc1, last1 = plsc.scan_count(g1)
                pos0 = plsc.load_gather(base0, [g0]) + rc0 - 1
                pos1 = plsc.load_gather(base1, [g1]) + rc1 - 1
                plsc.store_scatter(d0_, [pos0], w0)
                plsc.store_scatter(d1_, [pos1], w1)
                plsc.addupdate_scatter(base0, [g0], rc0, mask=last0)
                plsc.addupdate_scatter(base1, [g1], rc1, mask=last1)
        hist_zero()
        for v in range(NV):
            g0 = _digit(wB0[pl.ds(16 * v, 16)], 24)
            g1 = _digit(wB1[pl.ds(16 * v, 16)], 24)
            plsc.addupdate_scatter(hist0, [g0], ones)
            plsc.addupdate_scatter(hist1, [g1], ones)
        scan_bases()
        for v in range(NV):
            w0 = wB0[pl.ds(16 * v, 16)]
            w1 = wB1[pl.ds(16 * v, 16)]
            g0 = _digit(w0, 24)
            g1 = _digit(w1, 24)
            rc0, last0 = plsc.scan_count(g0)
            rc1, last1 = plsc.scan_count(g1)
            pos0 = plsc.load_gather(base0, [g0]) + rc0 - 1
            pos1 = plsc.load_gather(base1, [g1]) + rc1 - 1
            plsc.addupdate_scatter(srow, [pos0 >> 1], w0 & jnp.int32(255))
            plsc.addupdate_scatter(srow, [pos1 >> 1], w1 & jnp.int32(255))
            plsc.addupdate_scatter(base0, [g0], rc0, mask=last0)
            plsc.addupdate_scatter(base1, [g1], rc1, mask=last1)

    def img_body(jj, _):
        img = wid * IMGS_PER_WORKER + jj
        pltpu.sync_copy(x_hbm.at[pl.ds(img * H, H)], ximg)

        def pair_body(hp, _):
            for m in range(WO // 16):
                srow[pl.ds(16 * m, 16)] = zeros
            sort_pair(2 * hp)
            for m in range(WO // 16):
                oimg[pl.ds(hp * WO + 16 * m, 16)] = (
                    srow[pl.ds(16 * m, 16)].astype(jnp.float32) * 0.25)
            return 0

        lax.fori_loop(0, HO, pair_body, 0)
        pltpu.sync_copy(oimg, out_hbm.at[img])
        return 0

    lax.fori_loop(0, IMGS_PER_WORKER, img_body, 0)


@functools.partial(
    pl.kernel,
    out_type=jax.ShapeDtypeStruct((IMGS, HO * WO), jnp.float32),
    mesh=_MESH,
    compiler_params=pltpu.CompilerParams(needs_layout_passes=False),
    scratch_types=[
        pltpu.VMEM((H, W), jnp.float32),
        pltpu.VMEM((HO * WO,), jnp.float32),
        pltpu.VMEM((WO,), jnp.int32),
        pltpu.VMEM((W,), jnp.int32),
        pltpu.VMEM((W,), jnp.int32),
        pltpu.VMEM((NB,), jnp.int32),
        pltpu.VMEM((NB,), jnp.int32),
        pltpu.VMEM((W,), jnp.int32),
        pltpu.VMEM((W,), jnp.int32),
        pltpu.VMEM((NB,), jnp.int32),
        pltpu.VMEM((NB,), jnp.int32),
    ],
)
def _sc_kernel(x_hbm, out_hbm, *scratch):
    _body(x_hbm, out_hbm, *scratch)


@jax.jit
def kernel(x):
    b, c, h, w = x.shape
    xf = x.reshape(b * c * h, w)
    out = _sc_kernel(xf)
    return out.reshape(b, c, HO, WO)


# speedup vs baseline: 2.0563x; 1.1794x over previous
import functools
import jax
import jax.numpy as jnp
from jax import lax
from jax.experimental import pallas as pl
from jax.experimental.pallas import tpu as pltpu, tpu_sc as plsc

H = 224
W = 224
HO = H // 2
WO = W // 2
NV = W // 16
NB = 256
NBV = NB // 16
IMGS = 768
IMGS_PER_WORKER = IMGS // 32

_MESH = plsc.VectorSubcoreMesh(core_axis_name="c", subcore_axis_name="s")


def _sortable_i32(xf):
    xb = plsc.bitcast(xf, jnp.int32)
    flip = (xb >> 31) | jnp.int32(-(2**31))
    return xb ^ flip


def _digit(w_i, shift):
    d = (plsc.bitcast(w_i, jnp.uint32) >> jnp.uint32(shift)) & jnp.uint32(NB - 1)
    return plsc.bitcast(d, jnp.int32)


NROWS = 4


def _body(x_hbm, out_hbm, ximg, oimg, srow0, srow1, *sets):
    wid = lax.axis_index("s") * 2 + lax.axis_index("c")
    iota = lax.iota(jnp.int32, 16)
    ones = jnp.ones((16,), jnp.int32)
    zeros = jnp.zeros((16,), jnp.int32)

    ctxs = [tuple(sets[4 * r : 4 * r + 4]) for r in range(NROWS)]
    srows = [srow0, srow0, srow1, srow1]

    def sort_group(h0):
        def hist_zero():
            for i in range(NBV):
                for (_, _, hist, _) in ctxs:
                    hist[pl.ds(16 * i, 16)] = zeros

        def scan_bases():
            carries = [jnp.int32(0)] * NROWS
            for i in range(NBV):
                hvs = [c[2][pl.ds(16 * i, 16)] for c in ctxs]
                cs = [plsc.cumsum(hv) for hv in hvs]
                for r, (_, _, _, base) in enumerate(ctxs):
                    base[pl.ds(16 * i, 16)] = cs[r] - hvs[r] + carries[r]
                carries = [carries[r] + cs[r][15] for r in range(NROWS)]

        hist_zero()
        for v in range(NV):
            keys = [
                _sortable_i32(ximg[h0 + r, pl.ds(16 * v, 16)])
                for r in range(NROWS)
            ]
            for r, (_, _, hist, _) in enumerate(ctxs):
                plsc.addupdate_scatter(hist, [_digit(keys[r], 0)], ones)
        scan_bases()
        for v in range(NV):
            lane = iota + jnp.int32(16 * v)
            keys = [
                _sortable_i32(ximg[h0 + r, pl.ds(16 * v, 16)])
                for r in range(NROWS)
            ]
            ds_ = [_digit(k, 0) for k in keys]
            ws = [(k & jnp.int32(-256)) | lane for k in keys]
            rcs = [plsc.scan_count(d) for d in ds_]
            poss = [
                plsc.load_gather(ctxs[r][3], [ds_[r]]) + rcs[r][0] - 1
                for r in range(NROWS)
            ]
            for r in range(NROWS):
                plsc.store_scatter(ctxs[r][1], [poss[r]], ws[r])
            for r in range(NROWS):
                plsc.addupdate_scatter(
                    ctxs[r][3], [ds_[r]], rcs[r][0], mask=rcs[r][1])
        for p, (si, di) in enumerate([(1, 0), (0, 1)], start=1):
            shift = 8 * p
            hist_zero()
            for v in range(NV):
                gs = [
                    _digit(ctxs[r][si][pl.ds(16 * v, 16)], shift)
                    for r in range(NROWS)
                ]
                for r in range(NROWS):
                    plsc.addupdate_scatter(ctxs[r][2], [gs[r]], ones)
            scan_bases()
            for v in range(NV):
                ws = [ctxs[r][si][pl.ds(16 * v, 16)] for r in range(NROWS)]
                gs = [_digit(w, shift) for w in ws]
                rcs = [plsc.scan_count(g) for g in gs]
                poss = [
                    plsc.load_gather(ctxs[r][3], [gs[r]]) + rcs[r][0] - 1
                    for r in range(NROWS)
                ]
                for r in range(NROWS):
                    plsc.store_scatter(ctxs[r][di], [poss[r]], ws[r])
                for r in range(NROWS):
                    plsc.addupdate_scatter(
                        ctxs[r][3], [gs[r]], rcs[r][0], mask=rcs[r][1])
        hist_zero()
        for v in range(NV):
            gs = [
                _digit(ctxs[r][1][pl.ds(16 * v, 16)], 24)
                for r in range(NROWS)
            ]
            for r in range(NROWS):
                plsc.addupdate_scatter(ctxs[r][2], [gs[r]], ones)
        scan_bases()
        for v in range(NV):
            ws = [ctxs[r][1][pl.ds(16 * v, 16)] for r in range(NROWS)]
            gs = [_digit(w, 24) for w in ws]
            rcs = [plsc.scan_count(g) for g in gs]
            poss = [
                plsc.load_gather(ctxs[r][3], [gs[r]]) + rcs[r][0] - 1
                for r in range(NROWS)
            ]
            for r in range(NROWS):
                plsc.addupdate_scatter(
                    srows[r], [poss[r] >> 1], ws[r] & jnp.int32(255))
            for r in range(NROWS):
                plsc.addupdate_scatter(
                    ctxs[r][3], [gs[r]], rcs[r][0], mask=rcs[r][1])

    def img_body(jj, _):
        img = wid * IMGS_PER_WORKER + jj
        pltpu.sync_copy(x_hbm.at[pl.ds(img * H, H)], ximg)

        def group_body(g, _):
            for m in range(WO // 16):
                srow0[pl.ds(16 * m, 16)] = zeros
                srow1[pl.ds(16 * m, 16)] = zeros
            sort_group(NROWS * g)
            for m in range(WO // 16):
                hp = (NROWS // 2) * g
                oimg[pl.ds(hp * WO + 16 * m, 16)] = (
                    srow0[pl.ds(16 * m, 16)].astype(jnp.float32) * 0.25)
                oimg[pl.ds((hp + 1) * WO + 16 * m, 16)] = (
                    srow1[pl.ds(16 * m, 16)].astype(jnp.float32) * 0.25)
            return 0

        lax.fori_loop(0, H // NROWS, group_body, 0)
        pltpu.sync_copy(oimg, out_hbm.at[img])
        return 0

    lax.fori_loop(0, IMGS_PER_WORKER, img_body, 0)


@functools.partial(
    pl.kernel,
    out_type=jax.ShapeDtypeStruct((IMGS, HO * WO), jnp.float32),
    mesh=_MESH,
    compiler_params=pltpu.CompilerParams(needs_layout_passes=False),
    scratch_types=[
        pltpu.VMEM((H, W), jnp.float32),
        pltpu.VMEM((HO * WO,), jnp.float32),
        pltpu.VMEM((WO,), jnp.int32),
        pltpu.VMEM((WO,), jnp.int32),
    ] + 4 * [
        pltpu.VMEM((W,), jnp.int32),
        pltpu.VMEM((W,), jnp.int32),
        pltpu.VMEM((NB,), jnp.int32),
        pltpu.VMEM((NB,), jnp.int32),
    ],
)
def _sc_kernel(x_hbm, out_hbm, *scratch):
    _body(x_hbm, out_hbm, *scratch)


@jax.jit
def kernel(x):
    b, c, h, w = x.shape
    xf = x.reshape(b * c * h, w)
    out = _sc_kernel(xf)
    return out.reshape(b, c, HO, WO)


# speedup vs baseline: 2.2534x; 1.0959x over previous
import functools
import jax
import jax.numpy as jnp
from jax import lax
from jax.experimental import pallas as pl
from jax.experimental.pallas import tpu as pltpu, tpu_sc as plsc

H = 224
W = 224
HO = H // 2
WO = W // 2
NV = W // 16
NB = 256
NBV = NB // 16
IMGS = 768
IMGS_PER_WORKER = IMGS // 32

_MESH = plsc.VectorSubcoreMesh(core_axis_name="c", subcore_axis_name="s")


def _sortable_i32(xf):
    xb = plsc.bitcast(xf, jnp.int32)
    flip = (xb >> 31) | jnp.int32(-(2**31))
    return xb ^ flip


def _digit(w_i, shift):
    d = (plsc.bitcast(w_i, jnp.uint32) >> jnp.uint32(shift)) & jnp.uint32(NB - 1)
    return plsc.bitcast(d, jnp.int32)


NROWS = 8


def _body(x_hbm, out_hbm, ximg, oimg, srow0, srow1, srow2, srow3, *sets):
    wid = lax.axis_index("s") * 2 + lax.axis_index("c")
    iota = lax.iota(jnp.int32, 16)
    ones = jnp.ones((16,), jnp.int32)
    zeros = jnp.zeros((16,), jnp.int32)

    ctxs = [tuple(sets[4 * r : 4 * r + 4]) for r in range(NROWS)]
    srows = [srow0, srow0, srow1, srow1, srow2, srow2, srow3, srow3]

    def sort_group(h0):
        def hist_zero():
            for i in range(NBV):
                for (_, _, hist, _) in ctxs:
                    hist[pl.ds(16 * i, 16)] = zeros

        def scan_bases():
            carries = [jnp.int32(0)] * NROWS
            for i in range(NBV):
                hvs = [c[2][pl.ds(16 * i, 16)] for c in ctxs]
                cs = [plsc.cumsum(hv) for hv in hvs]
                for r, (_, _, _, base) in enumerate(ctxs):
                    base[pl.ds(16 * i, 16)] = cs[r] - hvs[r] + carries[r]
                carries = [carries[r] + cs[r][15] for r in range(NROWS)]

        hist_zero()
        for v in range(NV):
            keys = [
                _sortable_i32(ximg[h0 + r, pl.ds(16 * v, 16)])
                for r in range(NROWS)
            ]
            for r, (_, _, hist, _) in enumerate(ctxs):
                plsc.addupdate_scatter(hist, [_digit(keys[r], 0)], ones)
        scan_bases()
        for v in range(NV):
            lane = iota + jnp.int32(16 * v)
            keys = [
                _sortable_i32(ximg[h0 + r, pl.ds(16 * v, 16)])
                for r in range(NROWS)
            ]
            ds_ = [_digit(k, 0) for k in keys]
            ws = [(k & jnp.int32(-256)) | lane for k in keys]
            rcs = [plsc.scan_count(d) for d in ds_]
            poss = [
                plsc.load_gather(ctxs[r][3], [ds_[r]]) + rcs[r][0] - 1
                for r in range(NROWS)
            ]
            for r in range(NROWS):
                plsc.store_scatter(ctxs[r][1], [poss[r]], ws[r])
            for r in range(NROWS):
                plsc.addupdate_scatter(
                    ctxs[r][3], [ds_[r]], rcs[r][0], mask=rcs[r][1])
        for p, (si, di) in enumerate([(1, 0), (0, 1)], start=1):
            shift = 8 * p
            hist_zero()
            for v in range(NV):
                gs = [
                    _digit(ctxs[r][si][pl.ds(16 * v, 16)], shift)
                    for r in range(NROWS)
                ]
                for r in range(NROWS):
                    plsc.addupdate_scatter(ctxs[r][2], [gs[r]], ones)
            scan_bases()
            for v in range(NV):
                ws = [ctxs[r][si][pl.ds(16 * v, 16)] for r in range(NROWS)]
                gs = [_digit(w, shift) for w in ws]
                rcs = [plsc.scan_count(g) for g in gs]
                poss = [
                    plsc.load_gather(ctxs[r][3], [gs[r]]) + rcs[r][0] - 1
                    for r in range(NROWS)
                ]
                for r in range(NROWS):
                    plsc.store_scatter(ctxs[r][di], [poss[r]], ws[r])
                for r in range(NROWS):
                    plsc.addupdate_scatter(
                        ctxs[r][3], [gs[r]], rcs[r][0], mask=rcs[r][1])
        hist_zero()
        for v in range(NV):
            gs = [
                _digit(ctxs[r][1][pl.ds(16 * v, 16)], 24)
                for r in range(NROWS)
            ]
            for r in range(NROWS):
                plsc.addupdate_scatter(ctxs[r][2], [gs[r]], ones)
        scan_bases()
        for v in range(NV):
            ws = [ctxs[r][1][pl.ds(16 * v, 16)] for r in range(NROWS)]
            gs = [_digit(w, 24) for w in ws]
            rcs = [plsc.scan_count(g) for g in gs]
            poss = [
                plsc.load_gather(ctxs[r][3], [gs[r]]) + rcs[r][0] - 1
                for r in range(NROWS)
            ]
            for r in range(NROWS):
                plsc.addupdate_scatter(
                    srows[r], [poss[r] >> 1], ws[r] & jnp.int32(255))
            for r in range(NROWS):
                plsc.addupdate_scatter(
                    ctxs[r][3], [gs[r]], rcs[r][0], mask=rcs[r][1])

    def img_body(jj, _):
        img = wid * IMGS_PER_WORKER + jj
        pltpu.sync_copy(x_hbm.at[pl.ds(img * H, H)], ximg)

        def group_body(g, _):
            for m in range(WO // 16):
                srow0[pl.ds(16 * m, 16)] = zeros
                srow1[pl.ds(16 * m, 16)] = zeros
                srow2[pl.ds(16 * m, 16)] = zeros
                srow3[pl.ds(16 * m, 16)] = zeros
            sort_group(NROWS * g)
            for m in range(WO // 16):
                hp = (NROWS // 2) * g
                for q, sr in enumerate((srow0, srow1, srow2, srow3)):
                    oimg[pl.ds((hp + q) * WO + 16 * m, 16)] = (
                        sr[pl.ds(16 * m, 16)].astype(jnp.float32) * 0.25)
            return 0

        lax.fori_loop(0, H // NROWS, group_body, 0)
        pltpu.sync_copy(oimg, out_hbm.at[img])
        return 0

    lax.fori_loop(0, IMGS_PER_WORKER, img_body, 0)


@functools.partial(
    pl.kernel,
    out_type=jax.ShapeDtypeStruct((IMGS, HO * WO), jnp.float32),
    mesh=_MESH,
    compiler_params=pltpu.CompilerParams(needs_layout_passes=False),
    scratch_types=[
        pltpu.VMEM((H, W), jnp.float32),
        pltpu.VMEM((HO * WO,), jnp.float32),
        pltpu.VMEM((WO,), jnp.int32),
        pltpu.VMEM((WO,), jnp.int32),
        pltpu.VMEM((WO,), jnp.int32),
        pltpu.VMEM((WO,), jnp.int32),
    ] + 8 * [
        pltpu.VMEM((W,), jnp.int32),
        pltpu.VMEM((W,), jnp.int32),
        pltpu.VMEM((NB,), jnp.int32),
        pltpu.VMEM((NB,), jnp.int32),
    ],
)
def _sc_kernel(x_hbm, out_hbm, *scratch):
    _body(x_hbm, out_hbm, *scratch)


@jax.jit
def kernel(x):
    b, c, h, w = x.shape
    xf = x.reshape(b * c * h, w)
    out = _sc_kernel(xf)
    return out.reshape(b, c, HO, WO)


# speedup vs baseline: 2.5656x; 1.1385x over previous
import functools
import jax
import jax.numpy as jnp
from jax import lax
from jax.experimental import pallas as pl
from jax.experimental.pallas import tpu as pltpu, tpu_sc as plsc

H = 224
W = 224
HO = H // 2
WO = W // 2
NV = W // 16
NB = 256
NBV = NB // 16
IMGS = 768
IMGS_PER_WORKER = IMGS // 32

_MESH = plsc.VectorSubcoreMesh(core_axis_name="c", subcore_axis_name="s")


def _sortable_i32(xf):
    xb = plsc.bitcast(xf, jnp.int32)
    flip = (xb >> 31) | jnp.int32(-(2**31))
    return xb ^ flip


def _digit(w_i, shift):
    d = (plsc.bitcast(w_i, jnp.uint32) >> jnp.uint32(shift)) & jnp.uint32(NB - 1)
    return plsc.bitcast(d, jnp.int32)


NROWS = 8


def _body(x_hbm, out_hbm, ximg, oimg, srow0, srow1, srow2, srow3, *sets):
    wid = lax.axis_index("s") * 2 + lax.axis_index("c")
    iota = lax.iota(jnp.int32, 16)
    ones = jnp.ones((16,), jnp.int32)
    zeros = jnp.zeros((16,), jnp.int32)

    ctxs = [tuple(sets[4 * r : 4 * r + 4]) for r in range(NROWS)]
    srows = [srow0, srow0, srow1, srow1, srow2, srow2, srow3, srow3]

    def sort_group(h0):
        def hist_zero():
            for i in range(NBV):
                for (_, _, hist, _) in ctxs:
                    hist[pl.ds(16 * i, 16)] = zeros

        def scan_bases():
            carries = [jnp.int32(0)] * NROWS
            for i in range(NBV):
                hvs = [c[2][pl.ds(16 * i, 16)] for c in ctxs]
                cs = [plsc.cumsum(hv) for hv in hvs]
                for r, (_, _, _, base) in enumerate(ctxs):
                    base[pl.ds(16 * i, 16)] = cs[r] - hvs[r] + carries[r]
                carries = [carries[r] + cs[r][15] for r in range(NROWS)]

        hist_zero()
        for v in range(NV):
            keys = [
                _sortable_i32(ximg[h0 + r, pl.ds(16 * v, 16)])
                for r in range(NROWS)
            ]
            for r, (_, _, hist, _) in enumerate(ctxs):
                plsc.addupdate_scatter(hist, [_digit(keys[r], 0)], ones)
        scan_bases()
        hist_zero()
        for v in range(NV):
            lane = iota + jnp.int32(16 * v)
            keys = [
                _sortable_i32(ximg[h0 + r, pl.ds(16 * v, 16)])
                for r in range(NROWS)
            ]
            ds_ = [_digit(k, 0) for k in keys]
            ws = [(k & jnp.int32(-256)) | lane for k in keys]
            rcs = [plsc.scan_count(d) for d in ds_]
            poss = [
                plsc.load_gather(ctxs[r][3], [ds_[r]]) + rcs[r][0] - 1
                for r in range(NROWS)
            ]
            for r in range(NROWS):
                plsc.store_scatter(ctxs[r][1], [poss[r]], ws[r])
            for r in range(NROWS):
                plsc.addupdate_scatter(
                    ctxs[r][3], [ds_[r]], rcs[r][0], mask=rcs[r][1])
            for r in range(NROWS):
                plsc.addupdate_scatter(ctxs[r][2], [_digit(ws[r], 8)], ones)
        for p, (si, di) in enumerate([(1, 0), (0, 1)], start=1):
            shift = 8 * p
            scan_bases()
            hist_zero()
            for v in range(NV):
                ws = [ctxs[r][si][pl.ds(16 * v, 16)] for r in range(NROWS)]
                gs = [_digit(w, shift) for w in ws]
                rcs = [plsc.scan_count(g) for g in gs]
                poss = [
                    plsc.load_gather(ctxs[r][3], [gs[r]]) + rcs[r][0] - 1
                    for r in range(NROWS)
                ]
                for r in range(NROWS):
                    plsc.store_scatter(ctxs[r][di], [poss[r]], ws[r])
                for r in range(NROWS):
                    plsc.addupdate_scatter(
                        ctxs[r][3], [gs[r]], rcs[r][0], mask=rcs[r][1])
                for r in range(NROWS):
                    plsc.addupdate_scatter(
                        ctxs[r][2], [_digit(ws[r], shift + 8)], ones)
        scan_bases()
        for v in range(NV):
            ws = [ctxs[r][1][pl.ds(16 * v, 16)] for r in range(NROWS)]
            gs = [_digit(w, 24) for w in ws]
            rcs = [plsc.scan_count(g) for g in gs]
            poss = [
                plsc.load_gather(ctxs[r][3], [gs[r]]) + rcs[r][0] - 1
                for r in range(NROWS)
            ]
            for r in range(NROWS):
                plsc.addupdate_scatter(
                    srows[r], [poss[r] >> 1], ws[r] & jnp.int32(255))
            for r in range(NROWS):
                plsc.addupdate_scatter(
                    ctxs[r][3], [gs[r]], rcs[r][0], mask=rcs[r][1])

    def img_body(jj, _):
        img = wid * IMGS_PER_WORKER + jj
        pltpu.sync_copy(x_hbm.at[pl.ds(img * H, H)], ximg)

        def group_body(g, _):
            for m in range(WO // 16):
                srow0[pl.ds(16 * m, 16)] = zeros
                srow1[pl.ds(16 * m, 16)] = zeros
                srow2[pl.ds(16 * m, 16)] = zeros
                srow3[pl.ds(16 * m, 16)] = zeros
            sort_group(NROWS * g)
            for m in range(WO // 16):
                hp = (NROWS // 2) * g
                for q, sr in enumerate((srow0, srow1, srow2, srow3)):
                    oimg[pl.ds((hp + q) * WO + 16 * m, 16)] = (
                        sr[pl.ds(16 * m, 16)].astype(jnp.float32) * 0.25)
            return 0

        lax.fori_loop(0, H // NROWS, group_body, 0)
        pltpu.sync_copy(oimg, out_hbm.at[img])
        return 0

    lax.fori_loop(0, IMGS_PER_WORKER, img_body, 0)


@functools.partial(
    pl.kernel,
    out_type=jax.ShapeDtypeStruct((IMGS, HO * WO), jnp.float32),
    mesh=_MESH,
    compiler_params=pltpu.CompilerParams(needs_layout_passes=False),
    scratch_types=[
        pltpu.VMEM((H, W), jnp.float32),
        pltpu.VMEM((HO * WO,), jnp.float32),
        pltpu.VMEM((WO,), jnp.int32),
        pltpu.VMEM((WO,), jnp.int32),
        pltpu.VMEM((WO,), jnp.int32),
        pltpu.VMEM((WO,), jnp.int32),
    ] + 8 * [
        pltpu.VMEM((W,), jnp.int32),
        pltpu.VMEM((W,), jnp.int32),
        pltpu.VMEM((NB,), jnp.int32),
        pltpu.VMEM((NB,), jnp.int32),
    ],
)
def _sc_kernel(x_hbm, out_hbm, *scratch):
    _body(x_hbm, out_hbm, *scratch)


@jax.jit
def kernel(x):
    b, c, h, w = x.shape
    xf = x.reshape(b * c * h, w)
    out = _sc_kernel(xf)
    return out.reshape(b, c, HO, WO)


# speedup vs baseline: 2.7083x; 1.0556x over previous
import functools
import jax
import jax.numpy as jnp
from jax import lax
from jax.experimental import pallas as pl
from jax.experimental.pallas import tpu as pltpu, tpu_sc as plsc

H = 224
W = 224
HO = H // 2
WO = W // 2
NV = W // 16
NB = 256
NBV = NB // 16
IMGS = 768
IMGS_PER_WORKER = IMGS // 32

_MESH = plsc.VectorSubcoreMesh(core_axis_name="c", subcore_axis_name="s")


def _sortable_i32(xf):
    xb = plsc.bitcast(xf, jnp.int32)
    flip = (xb >> 31) | jnp.int32(-(2**31))
    return xb ^ flip


def _digit(w_i, shift):
    d = (plsc.bitcast(w_i, jnp.uint32) >> jnp.uint32(shift)) & jnp.uint32(NB - 1)
    return plsc.bitcast(d, jnp.int32)


NROWS = 14


def _body(x_hbm, out_hbm, ximg, oimg, *rest):
    srow_list = list(rest[: NROWS // 2])
    sets = rest[NROWS // 2 :]
    wid = lax.axis_index("s") * 2 + lax.axis_index("c")
    iota = lax.iota(jnp.int32, 16)
    ones = jnp.ones((16,), jnp.int32)
    zeros = jnp.zeros((16,), jnp.int32)

    ctxs = [tuple(sets[4 * r : 4 * r + 4]) for r in range(NROWS)]
    srows = [srow_list[r // 2] for r in range(NROWS)]

    def sort_group(h0):
        def hist_zero():
            for i in range(NBV):
                for (_, _, hist, _) in ctxs:
                    hist[pl.ds(16 * i, 16)] = zeros

        def scan_bases():
            carries = [jnp.int32(0)] * NROWS
            for i in range(NBV):
                hvs = [c[2][pl.ds(16 * i, 16)] for c in ctxs]
                cs = [plsc.cumsum(hv) for hv in hvs]
                for r, (_, _, _, base) in enumerate(ctxs):
                    base[pl.ds(16 * i, 16)] = cs[r] - hvs[r] + carries[r]
                carries = [carries[r] + cs[r][15] for r in range(NROWS)]

        hist_zero()
        for v in range(NV):
            keys = [
                _sortable_i32(ximg[h0 + r, pl.ds(16 * v, 16)])
                for r in range(NROWS)
            ]
            for r, (_, _, hist, _) in enumerate(ctxs):
                plsc.addupdate_scatter(hist, [_digit(keys[r], 0)], ones)
        scan_bases()
        hist_zero()
        for v in range(NV):
            lane = iota + jnp.int32(16 * v)
            keys = [
                _sortable_i32(ximg[h0 + r, pl.ds(16 * v, 16)])
                for r in range(NROWS)
            ]
            ds_ = [_digit(k, 0) for k in keys]
            ws = [(k & jnp.int32(-256)) | lane for k in keys]
            rcs = [plsc.scan_count(d) for d in ds_]
            poss = [
                plsc.load_gather(ctxs[r][3], [ds_[r]]) + rcs[r][0] - 1
                for r in range(NROWS)
            ]
            for r in range(NROWS):
                plsc.store_scatter(ctxs[r][1], [poss[r]], ws[r])
            for r in range(NROWS):
                plsc.addupdate_scatter(
                    ctxs[r][3], [ds_[r]], rcs[r][0], mask=rcs[r][1])
            for r in range(NROWS):
                plsc.addupdate_scatter(ctxs[r][2], [_digit(ws[r], 8)], ones)
        for p, (si, di) in enumerate([(1, 0), (0, 1)], start=1):
            shift = 8 * p
            scan_bases()
            hist_zero()
            for v in range(NV):
                ws = [ctxs[r][si][pl.ds(16 * v, 16)] for r in range(NROWS)]
                gs = [_digit(w, shift) for w in ws]
                rcs = [plsc.scan_count(g) for g in gs]
                poss = [
                    plsc.load_gather(ctxs[r][3], [gs[r]]) + rcs[r][0] - 1
                    for r in range(NROWS)
                ]
                for r in range(NROWS):
                    plsc.store_scatter(ctxs[r][di], [poss[r]], ws[r])
                for r in range(NROWS):
                    plsc.addupdate_scatter(
                        ctxs[r][3], [gs[r]], rcs[r][0], mask=rcs[r][1])
                for r in range(NROWS):
                    plsc.addupdate_scatter(
                        ctxs[r][2], [_digit(ws[r], shift + 8)], ones)
        scan_bases()
        for v in range(NV):
            ws = [ctxs[r][1][pl.ds(16 * v, 16)] for r in range(NROWS)]
            gs = [_digit(w, 24) for w in ws]
            rcs = [plsc.scan_count(g) for g in gs]
            poss = [
                plsc.load_gather(ctxs[r][3], [gs[r]]) + rcs[r][0] - 1
                for r in range(NROWS)
            ]
            for r in range(NROWS):
                plsc.addupdate_scatter(
                    srows[r], [poss[r] >> 1], ws[r] & jnp.int32(255))
            for r in range(NROWS):
                plsc.addupdate_scatter(
                    ctxs[r][3], [gs[r]], rcs[r][0], mask=rcs[r][1])

    def img_body(jj, _):
        img = wid * IMGS_PER_WORKER + jj
        pltpu.sync_copy(x_hbm.at[pl.ds(img * H, H)], ximg)

        def group_body(g, _):
            for m in range(WO // 16):
                for sr in srow_list:
                    sr[pl.ds(16 * m, 16)] = zeros
            sort_group(NROWS * g)
            for m in range(WO // 16):
                hp = (NROWS // 2) * g
                for q, sr in enumerate(srow_list):
                    oimg[pl.ds((hp + q) * WO + 16 * m, 16)] = (
                        sr[pl.ds(16 * m, 16)].astype(jnp.float32) * 0.25)
            return 0

        lax.fori_loop(0, H // NROWS, group_body, 0)
        pltpu.sync_copy(oimg, out_hbm.at[img])
        return 0

    lax.fori_loop(0, IMGS_PER_WORKER, img_body, 0)


@functools.partial(
    pl.kernel,
    out_type=jax.ShapeDtypeStruct((IMGS, HO * WO), jnp.float32),
    mesh=_MESH,
    compiler_params=pltpu.CompilerParams(needs_layout_passes=False),
    scratch_types=[
        pltpu.VMEM((H, W), jnp.float32),
        pltpu.VMEM((HO * WO,), jnp.float32),
    ] + (14 // 2) * [
        pltpu.VMEM((WO,), jnp.int32),
    ] + 14 * [
        pltpu.VMEM((W,), jnp.int32),
        pltpu.VMEM((W,), jnp.int32),
        pltpu.VMEM((NB,), jnp.int32),
        pltpu.VMEM((NB,), jnp.int32),
    ],
)
def _sc_kernel(x_hbm, out_hbm, *scratch):
    _body(x_hbm, out_hbm, *scratch)


@jax.jit
def kernel(x):
    b, c, h, w = x.shape
    xf = x.reshape(b * c * h, w)
    out = _sc_kernel(xf)
    return out.reshape(b, c, HO, WO)
